# baseline (speedup 1.0000x reference)
import functools

import jax
import jax.numpy as jnp
from jax import lax
from jax.experimental import pallas as pl
from jax.experimental.pallas import tpu as pltpu
from jax.experimental.pallas import tpu_sc as plsc

_BATCH = 16384
_DIM = 64
_NC = 2
_NS = 16
_NW = _NC * _NS
_PER_W = _BATCH // _NW
_CHUNK = 256
_NCHUNK = _PER_W // _CHUNK
_L = 16
_MARGIN = 1.0


def _vsqrt(x):
    i = lax.bitcast_convert_type(x, jnp.int32)
    y = lax.bitcast_convert_type(jnp.int32(0x5F3759DF) - (i >> 1), jnp.float32)
    xh = x * 0.5
    y = y * (1.5 - xh * y * y)
    y = y * (1.5 - xh * y * y)
    y = y * (1.5 - xh * y * y)
    return x * y


def _make_sc_call(interpret=False):
    mesh = plsc.VectorSubcoreMesh(
        core_axis_name="c", subcore_axis_name="s", num_cores=_NC, num_subcores=_NS
    )

    @functools.partial(
        pl.kernel,
        mesh=mesh,
        out_type=jax.ShapeDtypeStruct((_NW, _L), jnp.float32),
        scratch_types=[
            pltpu.VMEM((_CHUNK,), jnp.int32),
            pltpu.VMEM((_CHUNK,), jnp.int32),
            pltpu.VMEM((_CHUNK,), jnp.int32),
            pltpu.VMEM((_CHUNK,), jnp.int32),
            pltpu.VMEM((_CHUNK,), jnp.int32),
            pltpu.VMEM((_CHUNK,), jnp.int32),
            pltpu.VMEM((_CHUNK, _DIM), jnp.float32),
            pltpu.VMEM((_CHUNK, _DIM), jnp.float32),
            pltpu.VMEM((_CHUNK, _DIM), jnp.float32),
            pltpu.VMEM((_CHUNK, _DIM), jnp.float32),
            pltpu.VMEM((_CHUNK, _DIM), jnp.float32),
            pltpu.VMEM((_CHUNK, _DIM), jnp.float32),
            pltpu.VMEM((_L,), jnp.float32),
            pltpu.SemaphoreType.DMA,
        ],
        compiler_params=pltpu.CompilerParams(
            needs_layout_passes=False, use_tc_tiling_on_sc=False
        ),
        interpret=interpret,
    )
    def sc_call(ph, pr, pt, nh, nr, nt, ent, rel, out,
                ph_i, pr_i, pt_i, nh_i, nr_i, nt_i,
                ph_r, pr_r, pt_r, nh_r, nr_r, nt_r, acc_v, sem):
        wid = lax.axis_index("s") * _NC + lax.axis_index("c")
        base = wid * _PER_W

        lane = lax.iota(jnp.int32, 16)

        def chunk_body(ci, acc):
            off = base + ci * _CHUNK
            pltpu.sync_copy(ph.at[pl.ds(off, _CHUNK)], ph_i)
            pltpu.sync_copy(pr.at[pl.ds(off, _CHUNK)], pr_i)
            pltpu.sync_copy(pt.at[pl.ds(off, _CHUNK)], pt_i)
            pltpu.sync_copy(nh.at[pl.ds(off, _CHUNK)], nh_i)
            pltpu.sync_copy(nr.at[pl.ds(off, _CHUNK)], nr_i)
            pltpu.sync_copy(nt.at[pl.ds(off, _CHUNK)], nt_i)
            g1 = pltpu.async_copy(ent.at[ph_i], ph_r, sem)
            g2 = pltpu.async_copy(rel.at[pr_i], pr_r, sem)
            g3 = pltpu.async_copy(ent.at[pt_i], pt_r, sem)
            g4 = pltpu.async_copy(ent.at[nh_i], nh_r, sem)
            g5 = pltpu.async_copy(rel.at[nr_i], nr_r, sem)
            g6 = pltpu.async_copy(ent.at[nt_i], nt_r, sem)
            g1.wait(); g2.wait(); g3.wait(); g4.wait(); g5.wait(); g6.wait()

            def group_body(g, acc_in):
                rows = g * 16 + lane
                pos_ssq = jnp.zeros((16,), jnp.float32)
                neg_ssq = jnp.zeros((16,), jnp.float32)
                for k in range(_DIM):
                    col = jnp.full((16,), k, jnp.int32)
                    d = (plsc.load_gather(ph_r, [rows, col])
                         + plsc.load_gather(pr_r, [rows, col])
                         - plsc.load_gather(pt_r, [rows, col]))
                    pos_ssq = pos_ssq + d * d
                    e = (plsc.load_gather(nh_r, [rows, col])
                         + plsc.load_gather(nr_r, [rows, col])
                         - plsc.load_gather(nt_r, [rows, col]))
                    neg_ssq = neg_ssq + e * e
                term = jnp.maximum(_MARGIN + _vsqrt(pos_ssq) - _vsqrt(neg_ssq), 0.0)
                return acc_in + term

            return lax.fori_loop(0, _CHUNK // 16, group_body, acc)

        acc = lax.fori_loop(0, _NCHUNK, chunk_body, jnp.zeros((16,), jnp.float32))
        acc_v[...] = acc
        pltpu.sync_copy(acc_v, out.at[wid])

    return sc_call


_sc_call = _make_sc_call()


def kernel(pos_head, pos_relation, pos_tail, neg_head, neg_relation, neg_tail,
           entity_embedding, relation_embedding):
    partials = _sc_call(pos_head, pos_relation, pos_tail, neg_head, neg_relation,
                        neg_tail, entity_embedding, relation_embedding)
    return jnp.sum(partials)
